# Initial kernel scaffold; baseline (speedup 1.0000x reference)
import jax
import jax.numpy as jnp
from jax.experimental import pallas as pl


def kernel(x, edge_index, edge_attr, graph_ids, Wp, bp, We1, be1, We2, be2, conv_bias, Wih, Whh, bih, bhh, W1, b1, W2, b2):
    raise NotImplementedError("write your pallas kernel here")


# speedup vs baseline: 1.0205x; 1.0205x over previous
import functools
import math

import jax
import jax.numpy as jnp
from jax import lax
from jax.experimental import pallas as pl
from jax.experimental.pallas import tpu as pltpu
from jax.experimental.pallas import tpu_sc as plsc

D = 32
D_EH = 128
DD = D * D

_INTERPRET = False


def _proj_body(x_ref, wp_ref, bp_ref, o_ref):
    o_ref[...] = jax.nn.relu(
        jnp.dot(x_ref[...], wp_ref[...], preferred_element_type=jnp.float32)
        + bp_ref[...])


def _edge_body(ea_ref, hs_ref, we1_ref, be1_ref, we2p_ref, be2p_ref,
               sred_ref, o_ref):
    eh = jax.nn.relu(
        jnp.dot(ea_ref[...], we1_ref[...], preferred_element_type=jnp.float32)
        + be1_ref[...])
    ewp = jnp.dot(eh, we2p_ref[...], preferred_element_type=jnp.float32)
    ewp = ewp + be2p_ref[...]
    hs = hs_ref[...]
    mult = jnp.concatenate([hs] * D, axis=1)
    wide = mult * ewp
    o_ref[...] = jnp.dot(wide, sred_ref[...], preferred_element_type=jnp.float32)


def _gru_body(agg_ref, h_ref, cb_ref,
              wir_ref, wiz_ref, win_ref, whr_ref, whz_ref, whn_ref,
              bir_ref, biz_ref, bin_ref, bhr_ref, bhz_ref, bhn_ref, o_ref):
    n_rows = h_ref.shape[0]
    m = jax.nn.relu(agg_ref[0, :n_rows, :] + agg_ref[1, :n_rows, :]
                    + cb_ref[...])
    h = h_ref[...]

    def mm(a, b_ref):
        return jnp.dot(a, b_ref[...], preferred_element_type=jnp.float32)

    def sig(t):
        return 1.0 / (1.0 + jnp.exp(-t))

    r = sig(mm(m, wir_ref) + bir_ref[...] + mm(h, whr_ref) + bhr_ref[...])
    z = sig(mm(m, wiz_ref) + biz_ref[...] + mm(h, whz_ref) + bhz_ref[...])
    n = jnp.tanh(mm(m, win_ref) + bin_ref[...]
                 + r * (mm(h, whn_ref) + bhn_ref[...]))
    o_ref[...] = (1.0 - z) * n + z * h


def _readout_body(h_ref, gid_ref, w1_ref, b1_ref, w2_ref, b2_ref, o_ref):
    ids = gid_ref[...]
    rowid = lax.broadcasted_iota(jnp.int32, (128, 1), 0)
    oh = (ids == rowid).astype(jnp.float32)
    gf = jnp.dot(oh, h_ref[...], preferred_element_type=jnp.float32)
    hid = jax.nn.relu(
        jnp.dot(gf, w1_ref[...], preferred_element_type=jnp.float32)
        + b1_ref[...])
    o_ref[...] = (jnp.dot(hid, w2_ref[...], preferred_element_type=jnp.float32)
                  + b2_ref[...])


def _run_proj(x, wp, bp):
    n = x.shape[0]
    return pl.pallas_call(
        _proj_body,
        out_shape=jax.ShapeDtypeStruct((n, D), jnp.float32),
        interpret=_INTERPRET,
    )(x, wp, bp)


def _run_edge(ea_p, hs, we1, be1, we2p, be2p, sred, block_e):
    ep = ea_p.shape[0]
    grid = (ep // block_e,)
    full = lambda *shape: pl.BlockSpec(shape, lambda i: (0,) * len(shape))
    return pl.pallas_call(
        _edge_body,
        grid=grid,
        in_specs=[
            pl.BlockSpec((block_e, ea_p.shape[1]), lambda i: (i, 0)),
            pl.BlockSpec((block_e, D), lambda i: (i, 0)),
            full(*we1.shape), full(*be1.shape),
            full(*we2p.shape), full(*be2p.shape), full(*sred.shape),
        ],
        out_specs=pl.BlockSpec((block_e, D), lambda i: (i, 0)),
        out_shape=jax.ShapeDtypeStruct((ep, D), jnp.float32),
        interpret=_INTERPRET,
    )(ea_p, hs, we1, be1, we2p, be2p, sred)


def _run_gru(agg_pair, h, cb, gw):
    n = h.shape[0]
    return pl.pallas_call(
        _gru_body,
        out_shape=jax.ShapeDtypeStruct((n, D), jnp.float32),
        interpret=_INTERPRET,
    )(agg_pair, h, cb, *gw)


def _run_readout(h, gid2d, w1, b1, w2, b2):
    return pl.pallas_call(
        _readout_body,
        out_shape=jax.ShapeDtypeStruct((128, w2.shape[1]), jnp.float32),
        interpret=_INTERPRET,
    )(h, gid2d, w1, b1, w2, b2)


def kernel(x, edge_index, edge_attr, graph_ids, Wp, bp, We1, be1, We2, be2,
           conv_bias, Wih, Whh, bih, bhh, W1, b1, W2, b2):
    n, d_in = x.shape
    e = edge_attr.shape[0]
    d_e = edge_attr.shape[1]
    num_graphs = 100
    t_steps = 3

    chunk = 128
    workers = 32
    ep = ((e + workers * chunk - 1) // (workers * chunk)) * (workers * chunk)
    npad = ((n + 1 + 15) // 16) * 16

    src = edge_index[0]
    dst = edge_index[1]
    src_p = jnp.pad(src, (0, ep - e)).astype(jnp.int32)
    dst_p = jnp.pad(dst, (0, ep - e), constant_values=n).astype(jnp.int32)
    ea_p = jnp.pad(edge_attr, ((0, ep - e), (0, 0)))

    del d_e, d_in
    we2p = We2.reshape(D_EH, D, D).transpose(0, 2, 1).reshape(D_EH, DD)
    be2p = be2.reshape(D, D).T.reshape(1, DD)
    sred = (jnp.arange(DD)[:, None] // D == jnp.arange(D)[None, :]
            ).astype(jnp.float32)
    be1_2 = be1.reshape(1, D_EH)
    bp_2 = bp.reshape(1, D)
    cb_2 = conv_bias.reshape(1, D)
    wir, wiz, win = Wih[:D].T, Wih[D:2 * D].T, Wih[2 * D:].T
    whr, whz, whn = Whh[:D].T, Whh[D:2 * D].T, Whh[2 * D:].T
    bir, biz, bin_ = (bih[:D].reshape(1, D), bih[D:2 * D].reshape(1, D),
                      bih[2 * D:].reshape(1, D))
    bhr, bhz, bhn = (bhh[:D].reshape(1, D), bhh[D:2 * D].reshape(1, D),
                     bhh[2 * D:].reshape(1, D))
    gw = (wir, wiz, win, whr, whz, whn, bir, biz, bin_, bhr, bhz, bhn)
    gid2d = graph_ids.reshape(1, n).astype(jnp.int32)

    block_e = 2048

    h = _run_proj(x, Wp, bp_2)
    for _ in range(t_steps):
        hs = jnp.take(h, src_p, axis=0)
        msg = _run_edge(ea_p, hs, We1, be1_2, we2p, be2p, sred, block_e)
        agg = jax.ops.segment_sum(msg, dst_p, num_segments=npad)
        agg_pair = jnp.stack([agg, jnp.zeros_like(agg)])
        h = _run_gru(agg_pair, h, cb_2, gw)
    out = _run_readout(h, gid2d, W1, b1.reshape(1, D), W2,
                       b2.reshape(1, W2.shape[1]))
    return out[:num_graphs]


# speedup vs baseline: 2.5015x; 2.4513x over previous
import functools
import math

import jax
import jax.numpy as jnp
from jax import lax
from jax.experimental import pallas as pl
from jax.experimental.pallas import tpu as pltpu
from jax.experimental.pallas import tpu_sc as plsc

D = 32
D_EH = 128
DD = D * D

_INTERPRET = False


_NC = 2
_NS = 16
_NW = _NC * _NS
_CHUNK = 128


def _sc_gather_body(h_hbm, idx_hbm, out_hbm, idx_v, row_v, sem):
    wid = lax.axis_index("s") * _NC + lax.axis_index("c")
    nchunks = idx_v.shape[0]
    base = wid * nchunks
    pltpu.sync_copy(idx_hbm.at[pl.ds(base, nchunks)], idx_v)

    def body(j, carry):
        pltpu.async_copy(h_hbm.at[idx_v.at[j]], row_v, sem).wait()
        pltpu.sync_copy(row_v, out_hbm.at[pl.ds((base + j) * _CHUNK, _CHUNK)])
        return carry

    lax.fori_loop(0, nchunks, body, 0)


def _run_sc_gather(h, idx2d, ep):
    mesh = plsc.VectorSubcoreMesh(core_axis_name="c", subcore_axis_name="s",
                                  num_cores=_NC, num_subcores=_NS)
    nchunks = ep // (_NW * _CHUNK)
    f = functools.partial(
        pl.kernel,
        out_type=jax.ShapeDtypeStruct((ep, D), jnp.float32),
        mesh=mesh,
        scratch_types=[
            pltpu.VMEM((nchunks, _CHUNK), jnp.int32),
            pltpu.VMEM((_CHUNK, D), jnp.float32),
            pltpu.SemaphoreType.DMA,
        ],
        compiler_params=pltpu.CompilerParams(use_tc_tiling_on_sc=False),
    )(_sc_gather_body)
    return f(h, idx2d)


def _sc_scatter_body(msg_hbm, idx_hbm, zeros_hbm, out_hbm, idx_v, row_v, accum):
    c = lax.axis_index("c")
    s = lax.axis_index("s")
    wid = s * _NC + c
    nchunks = idx_v.shape[0]
    base = wid * nchunks
    rows_per = accum.shape[0] // _NS
    pltpu.sync_copy(zeros_hbm.at[pl.ds(s * rows_per, rows_per)],
                    accum.at[pl.ds(s * rows_per, rows_per)])
    pltpu.sync_copy(idx_hbm.at[pl.ds(base, nchunks)], idx_v)
    plsc.subcore_barrier()

    def body(j, carry):
        pltpu.sync_copy(msg_hbm.at[pl.ds((base + j) * _CHUNK, _CHUNK)], row_v)
        pltpu.sync_copy(row_v, accum.at[idx_v.at[j]], add=True)
        return carry

    lax.fori_loop(0, nchunks, body, 0)
    plsc.subcore_barrier()
    pltpu.sync_copy(accum.at[pl.ds(s * rows_per, rows_per)],
                    out_hbm.at[c].at[pl.ds(s * rows_per, rows_per)])


def _run_sc_scatter(msg, idx2d, zeros, npad):
    ep = msg.shape[0]
    mesh = plsc.VectorSubcoreMesh(core_axis_name="c", subcore_axis_name="s",
                                  num_cores=_NC, num_subcores=_NS)
    nchunks = ep // (_NW * _CHUNK)
    f = functools.partial(
        pl.kernel,
        out_type=jax.ShapeDtypeStruct((_NC, npad, D), jnp.float32),
        mesh=mesh,
        scratch_types=[
            pltpu.VMEM((nchunks, _CHUNK), jnp.int32),
            pltpu.VMEM((_CHUNK, D), jnp.float32),
            pltpu.VMEM_SHARED((npad, D), jnp.float32),
        ],
        compiler_params=pltpu.CompilerParams(use_tc_tiling_on_sc=False),
    )(_sc_scatter_body)
    return f(msg, idx2d, zeros)


def _proj_body(x_ref, wp_ref, bp_ref, o_ref):
    o_ref[...] = jax.nn.relu(
        jnp.dot(x_ref[...], wp_ref[...], preferred_element_type=jnp.float32)
        + bp_ref[...])


def _edge_body(ea_ref, hs_ref, we1_ref, be1_ref, we2p_ref, be2p_ref,
               sred_ref, o_ref):
    eh = jax.nn.relu(
        jnp.dot(ea_ref[...], we1_ref[...], preferred_element_type=jnp.float32)
        + be1_ref[...])
    ewp = jnp.dot(eh, we2p_ref[...], preferred_element_type=jnp.float32)
    ewp = ewp + be2p_ref[...]
    hs = hs_ref[...]
    mult = jnp.concatenate([hs] * D, axis=1)
    wide = mult * ewp
    o_ref[...] = jnp.dot(wide, sred_ref[...], preferred_element_type=jnp.float32)


def _gru_body(agg_ref, h_ref, cb_ref,
              wir_ref, wiz_ref, win_ref, whr_ref, whz_ref, whn_ref,
              bir_ref, biz_ref, bin_ref, bhr_ref, bhz_ref, bhn_ref, o_ref):
    n_rows = h_ref.shape[0]
    m = jax.nn.relu(agg_ref[0, :n_rows, :] + agg_ref[1, :n_rows, :]
                    + cb_ref[...])
    h = h_ref[...]

    def mm(a, b_ref):
        return jnp.dot(a, b_ref[...], preferred_element_type=jnp.float32)

    def sig(t):
        return 1.0 / (1.0 + jnp.exp(-t))

    r = sig(mm(m, wir_ref) + bir_ref[...] + mm(h, whr_ref) + bhr_ref[...])
    z = sig(mm(m, wiz_ref) + biz_ref[...] + mm(h, whz_ref) + bhz_ref[...])
    n = jnp.tanh(mm(m, win_ref) + bin_ref[...]
                 + r * (mm(h, whn_ref) + bhn_ref[...]))
    o_ref[...] = (1.0 - z) * n + z * h


def _readout_body(h_ref, gid_ref, w1_ref, b1_ref, w2_ref, b2_ref, o_ref):
    ids = gid_ref[...]
    rowid = lax.broadcasted_iota(jnp.int32, (128, 1), 0)
    oh = (ids == rowid).astype(jnp.float32)
    gf = jnp.dot(oh, h_ref[...], preferred_element_type=jnp.float32)
    hid = jax.nn.relu(
        jnp.dot(gf, w1_ref[...], preferred_element_type=jnp.float32)
        + b1_ref[...])
    o_ref[...] = (jnp.dot(hid, w2_ref[...], preferred_element_type=jnp.float32)
                  + b2_ref[...])


def _run_proj(x, wp, bp):
    n = x.shape[0]
    return pl.pallas_call(
        _proj_body,
        out_shape=jax.ShapeDtypeStruct((n, D), jnp.float32),
        interpret=_INTERPRET,
    )(x, wp, bp)


def _run_edge(ea_p, hs, we1, be1, we2p, be2p, sred, block_e):
    ep = ea_p.shape[0]
    grid = (ep // block_e,)
    full = lambda *shape: pl.BlockSpec(shape, lambda i: (0,) * len(shape))
    return pl.pallas_call(
        _edge_body,
        grid=grid,
        in_specs=[
            pl.BlockSpec((block_e, ea_p.shape[1]), lambda i: (i, 0)),
            pl.BlockSpec((block_e, D), lambda i: (i, 0)),
            full(*we1.shape), full(*be1.shape),
            full(*we2p.shape), full(*be2p.shape), full(*sred.shape),
        ],
        out_specs=pl.BlockSpec((block_e, D), lambda i: (i, 0)),
        out_shape=jax.ShapeDtypeStruct((ep, D), jnp.float32),
        interpret=_INTERPRET,
    )(ea_p, hs, we1, be1, we2p, be2p, sred)


def _run_gru(agg_pair, h, cb, gw):
    n = h.shape[0]
    return pl.pallas_call(
        _gru_body,
        out_shape=jax.ShapeDtypeStruct((n, D), jnp.float32),
        interpret=_INTERPRET,
    )(agg_pair, h, cb, *gw)


def _run_readout(h, gid2d, w1, b1, w2, b2):
    return pl.pallas_call(
        _readout_body,
        out_shape=jax.ShapeDtypeStruct((128, w2.shape[1]), jnp.float32),
        interpret=_INTERPRET,
    )(h, gid2d, w1, b1, w2, b2)


def kernel(x, edge_index, edge_attr, graph_ids, Wp, bp, We1, be1, We2, be2,
           conv_bias, Wih, Whh, bih, bhh, W1, b1, W2, b2):
    n, d_in = x.shape
    e = edge_attr.shape[0]
    d_e = edge_attr.shape[1]
    num_graphs = 100
    t_steps = 3

    chunk = 128
    workers = 32
    ep = ((e + workers * chunk - 1) // (workers * chunk)) * (workers * chunk)
    npad = ((n + 1 + 15) // 16) * 16

    src = edge_index[0]
    dst = edge_index[1]
    src_p = jnp.pad(src, (0, ep - e)).astype(jnp.int32)
    dst_p = jnp.pad(dst, (0, ep - e), constant_values=n).astype(jnp.int32)
    ea_p = jnp.pad(edge_attr, ((0, ep - e), (0, 0)))

    del d_e, d_in
    we2p = We2.reshape(D_EH, D, D).transpose(0, 2, 1).reshape(D_EH, DD)
    be2p = be2.reshape(D, D).T.reshape(1, DD)
    sred = (jnp.arange(DD)[:, None] // D == jnp.arange(D)[None, :]
            ).astype(jnp.float32)
    be1_2 = be1.reshape(1, D_EH)
    bp_2 = bp.reshape(1, D)
    cb_2 = conv_bias.reshape(1, D)
    wir, wiz, win = Wih[:D].T, Wih[D:2 * D].T, Wih[2 * D:].T
    whr, whz, whn = Whh[:D].T, Whh[D:2 * D].T, Whh[2 * D:].T
    bir, biz, bin_ = (bih[:D].reshape(1, D), bih[D:2 * D].reshape(1, D),
                      bih[2 * D:].reshape(1, D))
    bhr, bhz, bhn = (bhh[:D].reshape(1, D), bhh[D:2 * D].reshape(1, D),
                     bhh[2 * D:].reshape(1, D))
    gw = (wir, wiz, win, whr, whz, whn, bir, biz, bin_, bhr, bhz, bhn)
    gid2d = graph_ids.reshape(1, n).astype(jnp.int32)

    block_e = 2048

    src2d = src_p.reshape(ep // _CHUNK, _CHUNK)
    dst2d = dst_p.reshape(ep // _CHUNK, _CHUNK)
    zeros = jnp.zeros((npad, D), jnp.float32)

    h = _run_proj(x, Wp, bp_2)
    for _ in range(t_steps):
        hs = _run_sc_gather(h, src2d, ep)
        msg = _run_edge(ea_p, hs, We1, be1_2, we2p, be2p, sred, block_e)
        agg_pair = _run_sc_scatter(msg, dst2d, zeros, npad)
        h = _run_gru(agg_pair, h, cb_2, gw)
    out = _run_readout(h, gid2d, W1, b1.reshape(1, D), W2,
                       b2.reshape(1, W2.shape[1]))
    return out[:num_graphs]
